# baseline (speedup 1.0000x reference)
import functools

import jax
import jax.numpy as jnp
from jax import lax
from jax.experimental import pallas as pl
from jax.experimental.pallas import tpu as pltpu
from jax.experimental.pallas import tpu_sc as plsc

_NRELS = 100000
_B = 16384
_L = 200
_C = 16

_NC = 2
_NS = 16
_NW = _NC * _NS
_N = _B * _L
_TR = _L // 8
_TRW = _B * 8
_HALF_TRW = _TRW // 2
_NH = _N // 2
_K = _HALF_TRW // _NW
_NCHUNK = _TR


def _sc_gather_body(h, tbl_hbm, idx_hbm, out_hbm, tbl_v,
                    idx_v0, idx_v1, out_v0, out_v1,
                    sem_t, si0, si1, so0, so1):
    wid = lax.axis_index("s") * _NC + lax.axis_index("c")
    in_base = h * _HALF_TRW + wid * _K
    out_base = wid * _K
    idx_bufs = (idx_v0, idx_v1)
    out_bufs = (out_v0, out_v1)
    sin = (si0, si1)
    sout = (so0, so1)

    def in_slice(ci):
        return idx_hbm.at[pl.ds(in_base + ci * _TRW, _K)]

    def out_slice(ci):
        return out_hbm.at[pl.ds(out_base + ci * _HALF_TRW, _K)]

    tcp = pltpu.make_async_copy(tbl_hbm, tbl_v, sem_t)
    tcp.start()
    for par in range(2):
        pltpu.make_async_copy(in_slice(par), idx_bufs[par], sin[par]).start()
    tcp.wait()

    def _gather_chunk(par):
        def vec(j, c2):
            base_w = j * 128
            ids = [
                idx_bufs[par][pl.ds(base_w + k * 16, 16)] for k in range(8)
            ]
            vals = [plsc.load_gather(tbl_v, [iv]) for iv in ids]
            for k in range(8):
                out_bufs[par][pl.ds(base_w + k * 16, 16)] = vals[k]
            return c2

        lax.fori_loop(0, _K // 128, vec, 0)

    def process(ci, par, first_use, prefetch):
        pltpu.make_async_copy(in_slice(ci), idx_bufs[par], sin[par]).wait()
        if not first_use:
            pltpu.make_async_copy(
                out_bufs[par], out_slice(ci - 2), sout[par]
            ).wait()
        _gather_chunk(par)
        pltpu.make_async_copy(out_bufs[par], out_slice(ci), sout[par]).start()

        if prefetch == "always":
            pltpu.make_async_copy(
                in_slice(ci + 2), idx_bufs[par], sin[par]
            ).start()
        elif prefetch == "auto":

            @pl.when(ci + 2 < _NCHUNK)
            def _():
                pltpu.make_async_copy(
                    in_slice(ci + 2), idx_bufs[par], sin[par]
                ).start()

    for ci in range(2):
        process(ci, ci, True, "always")

    def pair(p, carry):
        process(p * 2, 0, False, "auto")
        process(p * 2 + 1, 1, False, "auto")
        return carry

    lax.fori_loop(1, (_NCHUNK - 1) // 2, pair, 0)
    process(_NCHUNK - 1, 0, False, "never")

    pltpu.make_async_copy(
        out_bufs[0], out_slice(_NCHUNK - 1), sout[0]
    ).wait()
    pltpu.make_async_copy(
        out_bufs[1], out_slice(_NCHUNK - 2), sout[1]
    ).wait()


def _make_sc_gather(h):
    return functools.partial(
        pl.kernel,
        mesh=plsc.VectorSubcoreMesh(core_axis_name="c", subcore_axis_name="s"),
        compiler_params=pltpu.CompilerParams(needs_layout_passes=False),
        out_type=jax.ShapeDtypeStruct((_NH,), jnp.float32),
        scratch_types=[
            pltpu.VMEM((_NRELS,), jnp.float32),
            pltpu.VMEM((_K,), jnp.int32),
            pltpu.VMEM((_K,), jnp.int32),
            pltpu.VMEM((_K,), jnp.float32),
            pltpu.VMEM((_K,), jnp.float32),
            pltpu.SemaphoreType.DMA,
            pltpu.SemaphoreType.DMA,
            pltpu.SemaphoreType.DMA,
            pltpu.SemaphoreType.DMA,
            pltpu.SemaphoreType.DMA,
        ],
    )(functools.partial(_sc_gather_body, h))


_sc_gather_halves = (_make_sc_gather(0), _make_sc_gather(1))


_W = 1024
_BH = _B // 2


def _tc_body(c_ref, x_ref, o_ref):
    def step(l, acc):
        return acc + x_ref[l] * c_ref[l][None, :]

    t = lax.fori_loop(
        0, _L, step, jnp.zeros((_C, _W), jnp.float32), unroll=8
    )
    m = jnp.max(t, axis=0, keepdims=True)
    e = jnp.exp(t - m)
    o_ref[...] = e / jnp.sum(e, axis=0, keepdims=True)


def _tile_flat(a2d):
    return a2d.reshape(_L // 8, 8, _B // 128, 128).transpose(0, 2, 1, 3).reshape(_N)


def _tile_unflat_half(flat):
    return (
        flat.reshape(_L // 8, _BH // 128, 8, 128)
        .transpose(0, 2, 1, 3)
        .reshape(_L, _BH)
    )


def kernel(rel_indices, x, d, b):
    del b
    xT = jnp.transpose(x, (1, 2, 0))
    relT = jnp.transpose(rel_indices, (1, 0))
    idx_flat = _tile_flat(relT)
    tbl = d.reshape(_NRELS)
    outs = []
    for h in range(2):
        cT_h = _tile_unflat_half(_sc_gather_halves[h](tbl, idx_flat))
        outs.append(
            pl.pallas_call(
                _tc_body,
                grid=(_BH // _W,),
                in_specs=[
                    pl.BlockSpec((_L, _W), lambda i: (0, i)),
                    pl.BlockSpec(
                        (_L, _C, _W),
                        functools.partial(
                            lambda hh, i: (0, 0, i + hh * (_BH // _W)), h
                        ),
                    ),
                ],
                out_specs=pl.BlockSpec((_C, _W), lambda i: (0, i)),
                out_shape=jax.ShapeDtypeStruct((_C, _BH), jnp.float32),
            )(cT_h, xT)
        )
    outT = jnp.concatenate(outs, axis=1)
    return jnp.transpose(outT, (1, 0))


# speedup vs baseline: 1.1412x; 1.1412x over previous
import functools

import jax
import jax.numpy as jnp
from jax import lax
from jax.experimental import pallas as pl
from jax.experimental.pallas import tpu as pltpu
from jax.experimental.pallas import tpu_sc as plsc

_NRELS = 100000
_B = 16384
_L = 200
_C = 16

_NC = 2
_NS = 16
_NW = _NC * _NS
_N = _B * _L
_NPW = _N // _NW
_K = 6400
_NCHUNK = _NPW // _K


def _sc_gather_body(
    tbl_hbm, idx_hbm, out_hbm, tbl_v,
    idx_v0, idx_v1, out_v0, out_v1,
    sem_t, si0, si1, so0, so1,
):
    wid = lax.axis_index("s") * _NC + lax.axis_index("c")
    base = wid * _NPW
    idx_bufs = (idx_v0, idx_v1)
    out_bufs = (out_v0, out_v1)
    sin = (si0, si1)
    sout = (so0, so1)

    tcp = pltpu.make_async_copy(tbl_hbm, tbl_v, sem_t)
    tcp.start()
    for par in range(2):
        pltpu.make_async_copy(
            idx_hbm.at[pl.ds(base + par * _K, _K)], idx_bufs[par], sin[par]
        ).start()
    tcp.wait()

    def _gather_chunk(par):
        def vec(j, c2):
            base_w = j * 128
            ids = [
                idx_bufs[par][pl.ds(base_w + k * 16, 16)] for k in range(8)
            ]
            vals = [plsc.load_gather(tbl_v, [iv]) for iv in ids]
            for k in range(8):
                out_bufs[par][pl.ds(base_w + k * 16, 16)] = vals[k]
            return c2

        lax.fori_loop(0, _K // 128, vec, 0)

    for par in range(2):
        off = base + par * _K
        pltpu.make_async_copy(
            idx_hbm.at[pl.ds(off, _K)], idx_bufs[par], sin[par]
        ).wait()
        _gather_chunk(par)
        pltpu.make_async_copy(
            out_bufs[par], out_hbm.at[pl.ds(off, _K)], sout[par]
        ).start()
        pltpu.make_async_copy(
            idx_hbm.at[pl.ds(off + 2 * _K, _K)], idx_bufs[par], sin[par]
        ).start()

    def pair(p, carry):
        for par in range(2):
            ci = p * 2 + par
            off = base + ci * _K
            pltpu.make_async_copy(
                idx_hbm.at[pl.ds(off, _K)], idx_bufs[par], sin[par]
            ).wait()
            pltpu.make_async_copy(
                out_bufs[par], out_hbm.at[pl.ds(off - 2 * _K, _K)], sout[par]
            ).wait()
            _gather_chunk(par)
            pltpu.make_async_copy(
                out_bufs[par], out_hbm.at[pl.ds(off, _K)], sout[par]
            ).start()

            @pl.when(ci + 2 < _NCHUNK)
            def _():
                pltpu.make_async_copy(
                    idx_hbm.at[pl.ds(off + 2 * _K, _K)], idx_bufs[par], sin[par]
                ).start()

        return carry

    lax.fori_loop(1, _NCHUNK // 2, pair, 0)

    for par in range(2):
        off = base + (_NCHUNK - 2 + par) * _K
        pltpu.make_async_copy(
            out_bufs[par], out_hbm.at[pl.ds(off, _K)], sout[par]
        ).wait()


_sc_gather = functools.partial(
    pl.kernel,
    mesh=plsc.VectorSubcoreMesh(core_axis_name="c", subcore_axis_name="s"),
    compiler_params=pltpu.CompilerParams(needs_layout_passes=False),
    out_type=jax.ShapeDtypeStruct((_N,), jnp.float32),
    scratch_types=[
        pltpu.VMEM((_NRELS,), jnp.float32),
        pltpu.VMEM((_K,), jnp.int32),
        pltpu.VMEM((_K,), jnp.int32),
        pltpu.VMEM((_K,), jnp.float32),
        pltpu.VMEM((_K,), jnp.float32),
        pltpu.SemaphoreType.DMA,
        pltpu.SemaphoreType.DMA,
        pltpu.SemaphoreType.DMA,
        pltpu.SemaphoreType.DMA,
        pltpu.SemaphoreType.DMA,
    ],
)(_sc_gather_body)


_W = 2048


def _tc_body(c_ref, x_ref, o_ref):
    def step(l, acc):
        return acc + x_ref[l] * c_ref[l][None, :]

    t = lax.fori_loop(
        0, _L, step, jnp.zeros((_C, _W), jnp.float32), unroll=8
    )
    m = jnp.max(t, axis=0, keepdims=True)
    e = jnp.exp(t - m)
    o_ref[...] = e / jnp.sum(e, axis=0, keepdims=True)


def _tile_flat(a2d):
    return a2d.reshape(_L // 8, 8, _B // 128, 128).transpose(0, 2, 1, 3).reshape(_N)


def _tile_unflat(flat):
    return (
        flat.reshape(_L // 8, _B // 128, 8, 128)
        .transpose(0, 2, 1, 3)
        .reshape(_L, _B)
    )


def kernel(rel_indices, x, d, b):
    del b
    xT = jnp.transpose(x, (1, 2, 0))
    relT = jnp.transpose(rel_indices, (1, 0))
    idx_flat = _tile_flat(relT)
    tbl = d.reshape(_NRELS)
    cT = _tile_unflat(_sc_gather(tbl, idx_flat))
    outT = pl.pallas_call(
        _tc_body,
        grid=(_B // _W,),
        in_specs=[
            pl.BlockSpec((_L, _W), lambda i: (0, i)),
            pl.BlockSpec((_L, _C, _W), lambda i: (0, 0, i)),
        ],
        out_specs=pl.BlockSpec((_C, _W), lambda i: (0, i)),
        out_shape=jax.ShapeDtypeStruct((_C, _B), jnp.float32),
    )(cT, xT)
    return jnp.transpose(outT, (1, 0))
